# Initial kernel scaffold; baseline (speedup 1.0000x reference)
import functools

import jax
import jax.numpy as jnp
from jax import lax
from jax.experimental import pallas as pl
from jax.experimental.pallas import tpu as pltpu
from jax.experimental.pallas import tpu_sc as plsc

NC = 2
NS = 16
NW = NC * NS

L = 16


def _sgpshift_body(B, C, T, V, x_hbm, s_hbm, out_hbm, idx_v, fidx_v, xbuf, obuf):
    TV = T * V
    PAIR = 2 * TV
    NPAIR = C // 2
    PPW = NPAIR // NW
    G = (PAIR + L - 1) // L
    GP = G * L
    CPW = 2 * PPW
    IDXW = CPW * V

    wid = lax.axis_index("s") * NC + lax.axis_index("c")

    pltpu.sync_copy(s_hbm.at[pl.ds(wid * IDXW, IDXW)], idx_v)

    @pl.loop(0, PPW * G)
    def _build(j):
        base = j * L
        p = base + lax.iota(jnp.int32, L)
        pair = p // GP
        q = p - pair * GP
        q = jnp.minimum(q, PAIR - 1)
        cl = q // TV
        r = q - cl * TV
        t = r // V
        v = r - t * V
        iv = plsc.load_gather(idx_v, [(pair * 2 + cl) * V + v])
        fidx_v[pl.ds(base, L)] = cl * TV + t * V + iv

    @pl.loop(0, B)
    def _batch(b):
        for pair in range(PPW):
            u = b * NPAIR + wid * PPW + pair
            pltpu.sync_copy(x_hbm.at[u], xbuf)

            @pl.loop(0, G)
            def _gather(j):
                base = j * L
                fi = fidx_v[pl.ds(pair * GP + base, L)]
                obuf[pl.ds(base, L)] = plsc.load_gather(xbuf, [fi])

            pltpu.sync_copy(obuf.at[pl.ds(0, PAIR)], out_hbm.at[u])


def kernel(x, shift_indices):
    B, C, T, V = x.shape
    TV = T * V
    PAIR = 2 * TV
    NPAIR = C // 2
    PPW = NPAIR // NW
    G = (PAIR + L - 1) // L
    GP = G * L

    x2 = x.reshape(B * NPAIR, PAIR)
    sflat = shift_indices.astype(jnp.int32).reshape(C * V)

    mesh = plsc.VectorSubcoreMesh(
        core_axis_name="c", subcore_axis_name="s", num_cores=NC, num_subcores=NS
    )
    run = pl.kernel(
        functools.partial(_sgpshift_body, B, C, T, V),
        out_type=jax.ShapeDtypeStruct((B * NPAIR, PAIR), jnp.float32),
        mesh=mesh,
        scratch_types=[
            pltpu.VMEM((2 * PPW * V,), jnp.int32),
            pltpu.VMEM((PPW * GP,), jnp.int32),
            pltpu.VMEM((PAIR,), jnp.float32),
            pltpu.VMEM((GP,), jnp.float32),
        ],
    )
    out = run(x2, sflat)
    return out.reshape(B, C, T, V)


# speedup vs baseline: 1.8566x; 1.8566x over previous
import functools

import jax
import jax.numpy as jnp
from jax import lax
from jax.experimental import pallas as pl
from jax.experimental.pallas import tpu as pltpu
from jax.experimental.pallas import tpu_sc as plsc

NC = 2
NS = 16
NW = NC * NS

L = 16


def _sgpshift_body(B, C, T, V, x_hbm, s_hbm, out_hbm, idx_v, fidx_v, xbuf, obuf):
    TV = T * V
    PAIR = 2 * TV
    NPAIR = C // 2
    PPW = NPAIR // NW
    G = (PAIR + L - 1) // L
    GP = G * L
    CPW = 2 * PPW
    IDXW = CPW * V

    wid = lax.axis_index("s") * NC + lax.axis_index("c")

    pltpu.sync_copy(s_hbm.at[pl.ds(wid * IDXW, IDXW)], idx_v)

    lanes = lax.iota(jnp.int32, L)
    zeros = jnp.full((L,), 0, jnp.int32)
    cV = jnp.full((L,), V, jnp.int32)
    cT = jnp.full((L,), T, jnp.int32)
    cTV = jnp.full((L,), TV, jnp.int32)
    cL = jnp.full((L,), L, jnp.int32)
    cIDXmax = jnp.full((L,), IDXW - 1, jnp.int32)
    cSRCmax = jnp.full((L,), PAIR - 1, jnp.int32)

    for pair in range(PPW):
        @pl.loop(0, G, init_carry=(zeros, zeros, lanes))
        def _build(j, carry, pair=pair):
            cl, t, v = carry
            gidx = jnp.minimum((pair * 2 + cl) * cV + v, cIDXmax)
            iv = plsc.load_gather(idx_v, [gidx])
            src = jnp.minimum(cl * cTV + t * cV + iv, cSRCmax)
            fidx_v[pl.ds(pair * GP + j * L, L)] = src
            v2 = v + cL
            ov = v2 >= cV
            v2 = jnp.where(ov, v2 - cV, v2)
            t2 = t + ov.astype(jnp.int32)
            ot = t2 >= cT
            t2 = jnp.where(ot, t2 - cT, t2)
            cl2 = cl + ot.astype(jnp.int32)
            return cl2, t2, v2

    GF = PAIR // L
    tailmask = lanes < jnp.full((L,), PAIR - GF * L, jnp.int32)

    @pl.loop(0, B)
    def _batch(b):
        for pair in range(PPW):
            u = b * NPAIR + wid * PPW + pair
            pltpu.sync_copy(x_hbm.at[u], xbuf)

            @pl.loop(0, GF)
            def _gather(j):
                base = j * L
                fi = fidx_v[pl.ds(pair * GP + base, L)]
                obuf[pl.ds(base, L)] = plsc.load_gather(xbuf, [fi])

            fi = fidx_v[pl.ds(pair * GP + GF * L, L)]
            val = plsc.load_gather(xbuf, [fi])
            oidx = jnp.minimum(jnp.full((L,), GF * L, jnp.int32) + lanes, cSRCmax)
            plsc.store_scatter(obuf, [oidx], val, mask=tailmask)

            pltpu.sync_copy(obuf, out_hbm.at[u])


def kernel(x, shift_indices):
    B, C, T, V = x.shape
    TV = T * V
    PAIR = 2 * TV
    NPAIR = C // 2
    PPW = NPAIR // NW
    G = (PAIR + L - 1) // L
    GP = G * L

    x2 = x.reshape(B * NPAIR, PAIR)
    sflat = shift_indices.astype(jnp.int32).reshape(C * V)

    mesh = plsc.VectorSubcoreMesh(
        core_axis_name="c", subcore_axis_name="s", num_cores=NC, num_subcores=NS
    )
    run = pl.kernel(
        functools.partial(_sgpshift_body, B, C, T, V),
        out_type=jax.ShapeDtypeStruct((B * NPAIR, PAIR), jnp.float32),
        mesh=mesh,
        scratch_types=[
            pltpu.VMEM((2 * PPW * V,), jnp.int32),
            pltpu.VMEM((PPW * GP,), jnp.int32),
            pltpu.VMEM((PAIR,), jnp.float32),
            pltpu.VMEM((PAIR,), jnp.float32),
        ],
        compiler_params=pltpu.CompilerParams(needs_layout_passes=False),
    )
    out = run(x2, sflat)
    return out.reshape(B, C, T, V)


# speedup vs baseline: 1.8633x; 1.0036x over previous
import functools

import jax
import jax.numpy as jnp
from jax import lax
from jax.experimental import pallas as pl
from jax.experimental.pallas import tpu as pltpu
from jax.experimental.pallas import tpu_sc as plsc

NC = 2
NS = 16
NW = NC * NS

L = 16


def _sgpshift_body(B, C, T, V, x_hbm, s_hbm, out_hbm, idx_v, fidx_v, xbuf, obuf):
    TV = T * V
    PAIR = 2 * TV
    NPAIR = C // 2
    PPW = NPAIR // NW
    G = (PAIR + L - 1) // L
    GP = G * L
    CPW = 2 * PPW
    IDXW = CPW * V

    wid = lax.axis_index("s") * NC + lax.axis_index("c")

    pltpu.sync_copy(s_hbm.at[pl.ds(wid * IDXW, IDXW)], idx_v)

    lanes = lax.iota(jnp.int32, L)
    zeros = jnp.full((L,), 0, jnp.int32)
    cV = jnp.full((L,), V, jnp.int32)
    cT = jnp.full((L,), T, jnp.int32)
    cTV = jnp.full((L,), TV, jnp.int32)
    cL = jnp.full((L,), L, jnp.int32)
    cIDXmax = jnp.full((L,), IDXW - 1, jnp.int32)
    cSRCmax = jnp.full((L,), PAIR - 1, jnp.int32)

    for pair in range(PPW):
        @pl.loop(0, G, init_carry=(zeros, zeros, lanes))
        def _build(j, carry, pair=pair):
            cl, t, v = carry
            gidx = jnp.minimum((pair * 2 + cl) * cV + v, cIDXmax)
            iv = plsc.load_gather(idx_v, [gidx])
            src = jnp.minimum(cl * cTV + t * cV + iv, cSRCmax)
            fidx_v[pl.ds(pair * GP + j * L, L)] = src
            v2 = v + cL
            ov = v2 >= cV
            v2 = jnp.where(ov, v2 - cV, v2)
            t2 = t + ov.astype(jnp.int32)
            ot = t2 >= cT
            t2 = jnp.where(ot, t2 - cT, t2)
            cl2 = cl + ot.astype(jnp.int32)
            return cl2, t2, v2

    GF = PAIR // L
    tailmask = lanes < jnp.full((L,), PAIR - GF * L, jnp.int32)

    @pl.loop(0, B)
    def _batch(b):
        for pair in range(PPW):
            u = b * NPAIR + wid * PPW + pair
            pltpu.sync_copy(x_hbm.at[u], xbuf)

            @pl.loop(0, GF, unroll=8)
            def _gather(j):
                base = j * L
                fi = fidx_v[pl.ds(pair * GP + base, L)]
                obuf[pl.ds(base, L)] = plsc.load_gather(xbuf, [fi])

            fi = fidx_v[pl.ds(pair * GP + GF * L, L)]
            val = plsc.load_gather(xbuf, [fi])
            oidx = jnp.minimum(jnp.full((L,), GF * L, jnp.int32) + lanes, cSRCmax)
            plsc.store_scatter(obuf, [oidx], val, mask=tailmask)

            pltpu.sync_copy(obuf, out_hbm.at[u])


def kernel(x, shift_indices):
    B, C, T, V = x.shape
    TV = T * V
    PAIR = 2 * TV
    NPAIR = C // 2
    PPW = NPAIR // NW
    G = (PAIR + L - 1) // L
    GP = G * L

    x2 = x.reshape(B * NPAIR, PAIR)
    sflat = shift_indices.astype(jnp.int32).reshape(C * V)

    mesh = plsc.VectorSubcoreMesh(
        core_axis_name="c", subcore_axis_name="s", num_cores=NC, num_subcores=NS
    )
    run = pl.kernel(
        functools.partial(_sgpshift_body, B, C, T, V),
        out_type=jax.ShapeDtypeStruct((B * NPAIR, PAIR), jnp.float32),
        mesh=mesh,
        scratch_types=[
            pltpu.VMEM((2 * PPW * V,), jnp.int32),
            pltpu.VMEM((PPW * GP,), jnp.int32),
            pltpu.VMEM((PAIR,), jnp.float32),
            pltpu.VMEM((PAIR,), jnp.float32),
        ],
        compiler_params=pltpu.CompilerParams(needs_layout_passes=False),
    )
    out = run(x2, sflat)
    return out.reshape(B, C, T, V)


# speedup vs baseline: 12.1265x; 6.5081x over previous
import functools

import jax
import jax.numpy as jnp
from jax import lax
from jax.experimental import pallas as pl
from jax.experimental.pallas import tpu as pltpu
from jax.experimental.pallas import tpu_sc as plsc

NC = 2
NS = 16
NW = NC * NS

L = 16
BB = 8
CB = 128


def _sgpshift_body(B, C, T, V, x_hbm, s_hbm, out_hbm, sbuf, itab, xbuf, obuf,
                   sem_in0, sem_in1, sem_out0, sem_out1):
    NBT = B // BB
    NCT = C // CB
    PLANE = BB * CB
    TPW = T * NBT * NCT // NW

    wid = lax.axis_index("s") * NC + lax.axis_index("c")
    ctile = wid & (NCT - 1)
    btile = (wid >> 1) & (NBT - 1)
    t0 = wid >> 3
    TSTRIDE = NW // (NBT * NCT)

    lanes = lax.iota(jnp.int32, L)
    cV = jnp.full((L,), V, jnp.int32)
    cPLANE = jnp.full((L,), PLANE, jnp.int32)

    pltpu.sync_copy(s_hbm.at[pl.ds(ctile * CB * V, CB * V)], sbuf)
    for v in range(V):
        @pl.loop(0, CB // L)
        def _tr(clb, v=v):
            cl = clb * L
            src = (jnp.full((L,), cl, jnp.int32) + lanes) * cV + jnp.full(
                (L,), v, jnp.int32)
            itab[pl.ds(v * CB + cl, L)] = plsc.load_gather(sbuf, [src])

    def unit_slices(hbm, k):
        t = t0 + k * TSTRIDE
        return hbm.at[pl.ds(t * V, V), pl.ds(btile * BB, BB), pl.ds(ctile * CB, CB)]

    def gather_unit(xb, ob):
        for v in range(V):
            @pl.loop(0, CB // L)
            def _g(clb, v=v):
                cl = clb * L
                clvec = jnp.full((L,), cl, jnp.int32) + lanes
                iv = itab[pl.ds(v * CB + cl, L)]
                for r in range(BB):
                    rvec = jnp.full((L,), r, jnp.int32)
                    ob[v, r, pl.ds(cl, L)] = plsc.load_gather(xb, [iv, rvec, clvec])

    def start_in(k, buf, sem):
        pltpu.async_copy(unit_slices(x_hbm, k), buf, sem)

    def start_out(k, buf, sem):
        pltpu.async_copy(buf, unit_slices(out_hbm, k), sem)

    def xb(i):
        return xbuf.at[i]

    def ob(i):
        return obuf.at[i]

    start_in(0, xb(0), sem_in0)

    @pl.loop(0, TPW, step=2)
    def _units(k):
        @pl.when(k + 1 < TPW)
        def _():
            start_in(k + 1, xb(1), sem_in1)

        pltpu.make_async_copy(unit_slices(x_hbm, k), xb(0), sem_in0).wait()

        @pl.when(k >= 2)
        def _():
            pltpu.make_async_copy(ob(0), unit_slices(out_hbm, k - 2), sem_out0).wait()

        gather_unit(xb(0), ob(0))
        start_out(k, ob(0), sem_out0)

        @pl.when(k + 1 < TPW)
        def _():
            @pl.when(k + 2 < TPW)
            def _():
                start_in(k + 2, xb(0), sem_in0)

            pltpu.make_async_copy(unit_slices(x_hbm, k + 1), xb(1), sem_in1).wait()

            @pl.when(k >= 1)
            def _():
                pltpu.make_async_copy(ob(1), unit_slices(out_hbm, k - 1), sem_out1).wait()

            gather_unit(xb(1), ob(1))
            start_out(k + 1, ob(1), sem_out1)

    if TPW >= 2:
        u = TPW - 2
        pltpu.make_async_copy(
            ob(u & 1), unit_slices(out_hbm, u), sem_out1 if (u & 1) else sem_out0
        ).wait()
    u = TPW - 1
    pltpu.make_async_copy(
        ob(u & 1), unit_slices(out_hbm, u), sem_out1 if (u & 1) else sem_out0
    ).wait()


def kernel(x, shift_indices):
    B, C, T, V = x.shape
    PLANE = BB * CB

    xt = jnp.transpose(x, (2, 3, 0, 1)).reshape(T * V, B, C)
    sflat = shift_indices.astype(jnp.int32).reshape(C * V)

    mesh = plsc.VectorSubcoreMesh(
        core_axis_name="c", subcore_axis_name="s", num_cores=NC, num_subcores=NS
    )
    run = pl.kernel(
        functools.partial(_sgpshift_body, B, C, T, V),
        out_type=jax.ShapeDtypeStruct((T * V, B, C), jnp.float32),
        mesh=mesh,
        scratch_types=[
            pltpu.VMEM((CB * V,), jnp.int32),
            pltpu.VMEM((V * CB,), jnp.int32),
            pltpu.VMEM((2, V, BB, CB), jnp.float32),
            pltpu.VMEM((2, V, BB, CB), jnp.float32),
            pltpu.SemaphoreType.DMA,
            pltpu.SemaphoreType.DMA,
            pltpu.SemaphoreType.DMA,
            pltpu.SemaphoreType.DMA,
        ],
        compiler_params=pltpu.CompilerParams(needs_layout_passes=False),
    )
    out_t = run(xt, sflat)
    return jnp.transpose(out_t.reshape(T, V, B, C), (2, 3, 0, 1))
